# Initial kernel scaffold; baseline (speedup 1.0000x reference)
import jax
import jax.numpy as jnp
from jax.experimental import pallas as pl


def kernel(x, edge_index, W1, b1, W2, b2):
    raise NotImplementedError("write your pallas kernel here")


# speedup vs baseline: 3.3572x; 3.3572x over previous
import functools

import jax
import jax.numpy as jnp
from jax import lax
from jax.experimental import pallas as pl
from jax.experimental.pallas import tpu as pltpu
from jax.experimental.pallas import tpu_sc as plsc

N = 10000
E = 320000
D = 128

NC = 2
NS = 16
NW = NC * NS

CH = 128
NCHUNK = 80
EPW = CH * NCHUNK
EPAD = EPW * NW

RPT = 632
RPAD = RPT * NS
DPT = 640
DPAD = DPT * NS
PAD_DST = N

ROWS_B = 1000


def _sc_agg_body(y, srci, dsti, p, dcnt, idx_s, idx_d, rows, ones_v, zed,
                 agg, deg):
    c = lax.axis_index("c")
    s = lax.axis_index("s")
    wid = s * NC + c

    z16 = jnp.zeros((16,), jnp.float32)

    def _zrow(i, carry):
        def _zcol(j, carry2):
            rows[i, pl.ds(j * 16, 16)] = z16
            return carry2
        return lax.fori_loop(0, D // 16, _zcol, carry)
    lax.fori_loop(0, CH, _zrow, 0)

    def _ones(j, carry):
        ones_v[pl.ds(j * 16, 16)] = z16 + 1.0
        return carry
    lax.fori_loop(0, CH // 16, _ones, 0)

    def _zed(j, carry):
        zed[pl.ds(j * 16, 16)] = z16
        return carry
    lax.fori_loop(0, DPT // 16, _zed, 0)

    base = s * RPT
    for k in range(RPT // CH):
        pltpu.sync_copy(rows, agg.at[pl.ds(base + k * CH, CH)])
    rem = RPT - (RPT // CH) * CH
    pltpu.sync_copy(rows.at[pl.ds(0, rem)],
                    agg.at[pl.ds(base + RPT - rem, rem)])
    pltpu.sync_copy(zed, deg.at[pl.ds(s * DPT, DPT)])

    pltpu.sync_copy(srci.at[wid], idx_s)
    pltpu.sync_copy(dsti.at[wid], idx_d)

    plsc.subcore_barrier()

    def _chunk(j, carry):
        pltpu.sync_copy(y.at[idx_s.at[j]], rows)
        pltpu.sync_copy(rows, agg.at[idx_d.at[j]], add=True)
        pltpu.sync_copy(ones_v, deg.at[idx_d.at[j]], add=True)
        return carry
    lax.fori_loop(0, NCHUNK, _chunk, 0)

    plsc.subcore_barrier()

    pltpu.sync_copy(agg.at[pl.ds(base, RPT)],
                    p.at[pl.ds(c * RPAD + base, RPT)])
    pltpu.sync_copy(deg.at[pl.ds(s * DPT, DPT)],
                    dcnt.at[pl.ds(c * DPAD + s * DPT, DPT)])


def _make_sc_agg():
    mesh = plsc.VectorSubcoreMesh(core_axis_name="c", subcore_axis_name="s",
                                  num_cores=NC, num_subcores=NS)
    return pl.kernel(
        _sc_agg_body,
        out_type=(jax.ShapeDtypeStruct((NC * RPAD, D), jnp.float32),
                  jax.ShapeDtypeStruct((NC * DPAD,), jnp.float32)),
        mesh=mesh,
        scratch_types=[
            pltpu.VMEM((NCHUNK, CH), jnp.int32),
            pltpu.VMEM((NCHUNK, CH), jnp.int32),
            pltpu.VMEM((CH, D), jnp.float32),
            pltpu.VMEM((CH,), jnp.float32),
            pltpu.VMEM((DPT,), jnp.float32),
            pltpu.VMEM_SHARED((RPAD, D), jnp.float32),
            pltpu.VMEM_SHARED((DPAD,), jnp.float32),
        ],
    )


def _mm_body(x_ref, w_ref, o_ref):
    o_ref[...] = jnp.dot(x_ref[...], w_ref[...],
                         preferred_element_type=jnp.float32)


def _tc_mm(x, w):
    return pl.pallas_call(
        _mm_body,
        grid=(N // ROWS_B,),
        in_specs=[pl.BlockSpec((ROWS_B, D), lambda i: (i, 0)),
                  pl.BlockSpec((D, D), lambda i: (0, 0))],
        out_specs=pl.BlockSpec((ROWS_B, D), lambda i: (i, 0)),
        out_shape=jax.ShapeDtypeStruct((N, D), jnp.float32),
    )(x, w)


def _norm_mm_body(p0_ref, p1_ref, d0_ref, d1_ref, b_ref, w_ref,
                  y_ref, r_ref):
    r = 1.0 / jnp.maximum(d0_ref[...] + d1_ref[...], 1.0)
    h = (p0_ref[...] + p1_ref[...]) * r + b_ref[...]
    y_ref[...] = jnp.dot(h, w_ref[...], preferred_element_type=jnp.float32)
    r_ref[...] = r


def _tc_norm_mm(p0, p1, d0, d1, b, w):
    return pl.pallas_call(
        _norm_mm_body,
        grid=(N // ROWS_B,),
        in_specs=[pl.BlockSpec((ROWS_B, D), lambda i: (i, 0)),
                  pl.BlockSpec((ROWS_B, D), lambda i: (i, 0)),
                  pl.BlockSpec((ROWS_B, 1), lambda i: (i, 0)),
                  pl.BlockSpec((ROWS_B, 1), lambda i: (i, 0)),
                  pl.BlockSpec((1, D), lambda i: (0, 0)),
                  pl.BlockSpec((D, D), lambda i: (0, 0))],
        out_specs=(pl.BlockSpec((ROWS_B, D), lambda i: (i, 0)),
                   pl.BlockSpec((ROWS_B, 1), lambda i: (i, 0))),
        out_shape=(jax.ShapeDtypeStruct((N, D), jnp.float32),
                   jax.ShapeDtypeStruct((N, 1), jnp.float32)),
    )(p0, p1, d0, d1, b, w)


def _final_body(q0_ref, q1_ref, r_ref, b_ref, o_ref):
    o_ref[...] = (q0_ref[...] + q1_ref[...]) * r_ref[...] + b_ref[...]


def _tc_final(q0, q1, r, b):
    return pl.pallas_call(
        _final_body,
        grid=(N // ROWS_B,),
        in_specs=[pl.BlockSpec((ROWS_B, D), lambda i: (i, 0)),
                  pl.BlockSpec((ROWS_B, D), lambda i: (i, 0)),
                  pl.BlockSpec((ROWS_B, 1), lambda i: (i, 0)),
                  pl.BlockSpec((1, D), lambda i: (0, 0))],
        out_specs=pl.BlockSpec((ROWS_B, D), lambda i: (i, 0)),
        out_shape=jax.ShapeDtypeStruct((N, D), jnp.float32),
    )(q0, q1, r, b)


def kernel(x, edge_index, W1, b1, W2, b2):
    src = edge_index[0].astype(jnp.int32)
    dst = edge_index[1].astype(jnp.int32)
    srcp = jnp.concatenate(
        [src, jnp.zeros((EPAD - E,), jnp.int32)]).reshape(NW, NCHUNK, CH)
    dstp = jnp.concatenate(
        [dst, jnp.full((EPAD - E,), PAD_DST, jnp.int32)]).reshape(NW, NCHUNK, CH)

    sc_agg = _make_sc_agg()

    y1 = _tc_mm(x, W1)
    p, dcnt = sc_agg(y1, srcp, dstp)
    p0 = p[:N]
    p1 = p[RPAD:RPAD + N]
    d0 = dcnt[:N].reshape(N, 1)
    d1 = dcnt[DPAD:DPAD + N].reshape(N, 1)

    y2, rdeg = _tc_norm_mm(p0, p1, d0, d1, b1.reshape(1, D), W2)

    q, _ = sc_agg(y2, srcp, dstp)
    q0 = q[:N]
    q1 = q[RPAD:RPAD + N]
    return _tc_final(q0, q1, rdeg, b2.reshape(1, D))


# speedup vs baseline: 3.6500x; 1.0872x over previous
import functools

import jax
import jax.numpy as jnp
from jax import lax
from jax.experimental import pallas as pl
from jax.experimental.pallas import tpu as pltpu
from jax.experimental.pallas import tpu_sc as plsc

N = 10000
E = 320000
D = 128

NC = 2
NS = 16
NW = NC * NS

CH = 128
NCHUNK = 80
EPW = CH * NCHUNK
EPAD = EPW * NW

RPT = 632
RPAD = RPT * NS
DPT = 640
DPAD = DPT * NS
PAD_DST = N
ISH = 14
IMASK = (1 << ISH) - 1

ROWS_B = 1000


def _unpack(pidx, j, buf, shift, mask):
    for t in range(CH // 16):
        v = pidx[j, pl.ds(t * 16, 16)]
        buf[pl.ds(t * 16, 16)] = (v >> shift) & mask


def _sc_agg_body(y, pidxh, p, dcnt, pidx, rows0, rows1, sbuf0, sbuf1, dbuf,
                 ones_v, zed, sem0, sem1, agg, deg):
    c = lax.axis_index("c")
    s = lax.axis_index("s")
    wid = s * NC + c

    z16 = jnp.zeros((16,), jnp.float32)

    def _zrow(i, carry):
        def _zcol(j, carry2):
            rows0[i, pl.ds(j * 16, 16)] = z16
            return carry2
        return lax.fori_loop(0, D // 16, _zcol, carry)
    lax.fori_loop(0, CH, _zrow, 0)

    for j in range(CH // 16):
        ones_v[pl.ds(j * 16, 16)] = z16 + 1.0

    def _zed(j, carry):
        zed[pl.ds(j * 16, 16)] = z16
        return carry
    lax.fori_loop(0, DPT // 16, _zed, 0)

    base = s * RPT
    for k in range(RPT // CH):
        pltpu.sync_copy(rows0, agg.at[pl.ds(base + k * CH, CH)])
    rem = RPT - (RPT // CH) * CH
    if rem:
        pltpu.sync_copy(rows0.at[pl.ds(0, rem)],
                        agg.at[pl.ds(base + RPT - rem, rem)])
    pltpu.sync_copy(zed, deg.at[pl.ds(s * DPT, DPT)])

    pltpu.sync_copy(pidxh.at[wid], pidx)

    plsc.subcore_barrier()

    _unpack(pidx, 0, sbuf0, 0, IMASK)
    pltpu.async_copy(y.at[sbuf0], rows0, sem0)

    def _pair(i, carry):
        j0 = 2 * i
        j1 = 2 * i + 1
        _unpack(pidx, j1, sbuf1, 0, IMASK)
        pltpu.make_async_copy(y.at[sbuf0], rows0, sem0).wait()
        pltpu.async_copy(y.at[sbuf1], rows1, sem1)

        _unpack(pidx, j0, dbuf, ISH, IMASK)
        pltpu.sync_copy(rows0, agg.at[dbuf], add=True)
        pltpu.sync_copy(ones_v, deg.at[dbuf], add=True)

        @pl.when(i < NCHUNK // 2 - 1)
        def _():
            _unpack(pidx, j0 + 2, sbuf0, 0, IMASK)

        pltpu.make_async_copy(y.at[sbuf1], rows1, sem1).wait()

        @pl.when(i < NCHUNK // 2 - 1)
        def _():
            pltpu.async_copy(y.at[sbuf0], rows0, sem0)

        _unpack(pidx, j1, dbuf, ISH, IMASK)
        pltpu.sync_copy(rows1, agg.at[dbuf], add=True)
        pltpu.sync_copy(ones_v, deg.at[dbuf], add=True)
        return carry
    lax.fori_loop(0, NCHUNK // 2, _pair, 0)

    plsc.subcore_barrier()

    pltpu.sync_copy(agg.at[pl.ds(base, RPT)],
                    p.at[pl.ds(c * RPAD + base, RPT)])
    pltpu.sync_copy(deg.at[pl.ds(s * DPT, DPT)],
                    dcnt.at[pl.ds(c * DPAD + s * DPT, DPT)])


def _make_sc_agg():
    mesh = plsc.VectorSubcoreMesh(core_axis_name="c", subcore_axis_name="s",
                                  num_cores=NC, num_subcores=NS)
    out_type = (jax.ShapeDtypeStruct((NC * RPAD, D), jnp.float32),
                jax.ShapeDtypeStruct((NC * DPAD,), jnp.float32))
    return pl.kernel(
        _sc_agg_body,
        out_type=out_type,
        mesh=mesh,
        scratch_types=[
            pltpu.VMEM((NCHUNK, CH), jnp.int32),
            pltpu.VMEM((CH, D), jnp.float32),
            pltpu.VMEM((CH, D), jnp.float32),
            pltpu.VMEM((CH,), jnp.int32),
            pltpu.VMEM((CH,), jnp.int32),
            pltpu.VMEM((CH,), jnp.int32),
            pltpu.VMEM((CH,), jnp.float32),
            pltpu.VMEM((DPT,), jnp.float32),
            pltpu.SemaphoreType.DMA,
            pltpu.SemaphoreType.DMA,
            pltpu.VMEM_SHARED((RPAD, D), jnp.float32),
            pltpu.VMEM_SHARED((DPAD,), jnp.float32),
        ],
    )


def _mm_body(x_ref, w_ref, o_ref):
    o_ref[...] = jnp.dot(x_ref[...], w_ref[...],
                         preferred_element_type=jnp.float32)


def _tc_mm(x, w):
    return pl.pallas_call(
        _mm_body,
        grid=(N // ROWS_B,),
        in_specs=[pl.BlockSpec((ROWS_B, D), lambda i: (i, 0)),
                  pl.BlockSpec((D, D), lambda i: (0, 0))],
        out_specs=pl.BlockSpec((ROWS_B, D), lambda i: (i, 0)),
        out_shape=jax.ShapeDtypeStruct((N, D), jnp.float32),
    )(x, w)


def _norm_mm_body(p0_ref, p1_ref, d0_ref, d1_ref, b_ref, w_ref,
                  y_ref, r_ref):
    r = 1.0 / jnp.maximum(d0_ref[...] + d1_ref[...], 1.0)
    h = (p0_ref[...] + p1_ref[...]) * r + b_ref[...]
    y_ref[...] = jnp.dot(h, w_ref[...], preferred_element_type=jnp.float32)
    r_ref[...] = r


def _tc_norm_mm(p0, p1, d0, d1, b, w):
    return pl.pallas_call(
        _norm_mm_body,
        grid=(N // ROWS_B,),
        in_specs=[pl.BlockSpec((ROWS_B, D), lambda i: (i, 0)),
                  pl.BlockSpec((ROWS_B, D), lambda i: (i, 0)),
                  pl.BlockSpec((ROWS_B, 1), lambda i: (i, 0)),
                  pl.BlockSpec((ROWS_B, 1), lambda i: (i, 0)),
                  pl.BlockSpec((1, D), lambda i: (0, 0)),
                  pl.BlockSpec((D, D), lambda i: (0, 0))],
        out_specs=(pl.BlockSpec((ROWS_B, D), lambda i: (i, 0)),
                   pl.BlockSpec((ROWS_B, 1), lambda i: (i, 0))),
        out_shape=(jax.ShapeDtypeStruct((N, D), jnp.float32),
                   jax.ShapeDtypeStruct((N, 1), jnp.float32)),
    )(p0, p1, d0, d1, b, w)


def _final_body(q0_ref, q1_ref, r_ref, b_ref, o_ref):
    o_ref[...] = (q0_ref[...] + q1_ref[...]) * r_ref[...] + b_ref[...]


def _tc_final(q0, q1, r, b):
    return pl.pallas_call(
        _final_body,
        grid=(N // ROWS_B,),
        in_specs=[pl.BlockSpec((ROWS_B, D), lambda i: (i, 0)),
                  pl.BlockSpec((ROWS_B, D), lambda i: (i, 0)),
                  pl.BlockSpec((ROWS_B, 1), lambda i: (i, 0)),
                  pl.BlockSpec((1, D), lambda i: (0, 0))],
        out_specs=pl.BlockSpec((ROWS_B, D), lambda i: (i, 0)),
        out_shape=jax.ShapeDtypeStruct((N, D), jnp.float32),
    )(q0, q1, r, b)


def kernel(x, edge_index, W1, b1, W2, b2):
    src = edge_index[0].astype(jnp.int32)
    dst = edge_index[1].astype(jnp.int32)
    packed = src | (dst << ISH)
    pidx = jnp.concatenate(
        [packed, jnp.full((EPAD - E,), PAD_DST << ISH, jnp.int32)]
    ).reshape(NW, NCHUNK, CH)

    sc_agg = _make_sc_agg()

    y1 = _tc_mm(x, W1)
    p, dcnt = sc_agg(y1, pidx)
    p0 = p[:N]
    p1 = p[RPAD:RPAD + N]
    d0 = dcnt[:N].reshape(N, 1)
    d1 = dcnt[DPAD:DPAD + N].reshape(N, 1)

    y2, rdeg = _tc_norm_mm(p0, p1, d0, d1, b1.reshape(1, D), W2)

    q, _ = sc_agg(y2, pidx)
    q0 = q[:N]
    q1 = q[RPAD:RPAD + N]
    return _tc_final(q0, q1, rdeg, b2.reshape(1, D))


# speedup vs baseline: 3.9349x; 1.0781x over previous
import functools

import jax
import jax.numpy as jnp
from jax import lax
from jax.experimental import pallas as pl
from jax.experimental.pallas import tpu as pltpu
from jax.experimental.pallas import tpu_sc as plsc

N = 10000
E = 320000
D = 128

NC = 2
NS = 16
NW = NC * NS

CH = 128
NCHUNKS = 2560
EPAD = NCHUNKS * CH
K0 = 120
K1 = 40

RPT = 632
RPAD = RPT * NS
DPT = 640
DPAD = DPT * NS
PAD_DST = N
ISH = 14
IMASK = (1 << ISH) - 1

ROWS_B = 1000


def _unpack(pidx, j, buf, shift, mask):
    for t in range(CH // 16):
        v = pidx[pl.ds(j * CH + t * 16, 16)]
        buf[pl.ds(t * 16, 16)] = (v >> shift) & mask


def _run_chunks(kc, y, pidx, rows0, rows1, sbuf0, sbuf1, dbuf, ones_v,
                sem0, sem1, agg, deg):
    _unpack(pidx, 0, sbuf0, 0, IMASK)
    pltpu.async_copy(y.at[sbuf0], rows0, sem0)

    def _pair(i, carry):
        j0 = 2 * i
        j1 = 2 * i + 1
        _unpack(pidx, j1, sbuf1, 0, IMASK)
        pltpu.make_async_copy(y.at[sbuf0], rows0, sem0).wait()
        pltpu.async_copy(y.at[sbuf1], rows1, sem1)

        _unpack(pidx, j0, dbuf, ISH, IMASK)
        pltpu.sync_copy(rows0, agg.at[dbuf], add=True)
        pltpu.sync_copy(ones_v, deg.at[dbuf], add=True)

        @pl.when(i < kc // 2 - 1)
        def _():
            _unpack(pidx, j0 + 2, sbuf0, 0, IMASK)

        pltpu.make_async_copy(y.at[sbuf1], rows1, sem1).wait()

        @pl.when(i < kc // 2 - 1)
        def _():
            pltpu.async_copy(y.at[sbuf0], rows0, sem0)

        _unpack(pidx, j1, dbuf, ISH, IMASK)
        pltpu.sync_copy(rows1, agg.at[dbuf], add=True)
        pltpu.sync_copy(ones_v, deg.at[dbuf], add=True)
        return carry
    lax.fori_loop(0, kc // 2, _pair, 0)


def _sc_agg_body(y, pidxh, p, dcnt, pidx, rows0, rows1, sbuf0, sbuf1, dbuf,
                 ones_v, sem0, sem1, agg, deg):
    c = lax.axis_index("c")
    s = lax.axis_index("s")

    z16 = jnp.zeros((16,), jnp.float32)

    def _zrow(i, carry):
        def _zcol(j, carry2):
            rows0[i, pl.ds(j * 16, 16)] = z16
            return carry2
        return lax.fori_loop(0, D // 16, _zcol, carry)
    lax.fori_loop(0, CH, _zrow, 0)

    for j in range(CH // 16):
        ones_v[pl.ds(j * 16, 16)] = z16 + 1.0

    base = s * RPT
    for k in range(RPT // CH):
        pltpu.sync_copy(rows0, agg.at[pl.ds(base + k * CH, CH)])
    rem = RPT - (RPT // CH) * CH
    if rem:
        pltpu.sync_copy(rows0.at[pl.ds(0, rem)],
                        agg.at[pl.ds(base + RPT - rem, rem)])
    for k in range(DPT // CH):
        pltpu.sync_copy(rows0.at[0], deg.at[pl.ds(s * DPT + k * CH, CH)])

    @pl.when(c == 0)
    def _():
        pltpu.sync_copy(pidxh.at[pl.ds(s * (K0 * CH), K0 * CH)],
                        pidx.at[pl.ds(0, K0 * CH)])

    @pl.when(c == 1)
    def _():
        pltpu.sync_copy(pidxh.at[pl.ds(NS * K0 * CH + s * (K1 * CH), K1 * CH)],
                        pidx.at[pl.ds(0, K1 * CH)])

    plsc.subcore_barrier()

    args = (y, pidx, rows0, rows1, sbuf0, sbuf1, dbuf, ones_v,
            sem0, sem1, agg, deg)

    @pl.when(c == 0)
    def _():
        _run_chunks(K0, *args)

    @pl.when(c == 1)
    def _():
        _run_chunks(K1, *args)

    plsc.subcore_barrier()

    pltpu.sync_copy(agg.at[pl.ds(base, RPT)],
                    p.at[pl.ds(c * RPAD + base, RPT)])
    pltpu.sync_copy(deg.at[pl.ds(s * DPT, DPT)],
                    dcnt.at[pl.ds(c * DPAD + s * DPT, DPT)])


def _make_sc_agg():
    mesh = plsc.VectorSubcoreMesh(core_axis_name="c", subcore_axis_name="s",
                                  num_cores=NC, num_subcores=NS)
    out_type = (jax.ShapeDtypeStruct((NC * RPAD, D), jnp.float32),
                jax.ShapeDtypeStruct((NC * DPAD,), jnp.float32))
    return pl.kernel(
        _sc_agg_body,
        out_type=out_type,
        mesh=mesh,
        scratch_types=[
            pltpu.VMEM((K0 * CH,), jnp.int32),
            pltpu.VMEM((CH, D), jnp.float32),
            pltpu.VMEM((CH, D), jnp.float32),
            pltpu.VMEM((CH,), jnp.int32),
            pltpu.VMEM((CH,), jnp.int32),
            pltpu.VMEM((CH,), jnp.int32),
            pltpu.VMEM((CH,), jnp.float32),
            pltpu.SemaphoreType.DMA,
            pltpu.SemaphoreType.DMA,
            pltpu.VMEM_SHARED((RPAD, D), jnp.float32),
            pltpu.VMEM_SHARED((DPAD,), jnp.float32),
        ],
    )


def _mm_body(x_ref, w_ref, o_ref):
    o_ref[...] = jnp.dot(x_ref[...], w_ref[...],
                         preferred_element_type=jnp.float32)


def _tc_mm(x, w):
    return pl.pallas_call(
        _mm_body,
        grid=(N // ROWS_B,),
        in_specs=[pl.BlockSpec((ROWS_B, D), lambda i: (i, 0)),
                  pl.BlockSpec((D, D), lambda i: (0, 0))],
        out_specs=pl.BlockSpec((ROWS_B, D), lambda i: (i, 0)),
        out_shape=jax.ShapeDtypeStruct((N, D), jnp.float32),
    )(x, w)


def _norm_mm_body(p0_ref, p1_ref, d0_ref, d1_ref, b_ref, w_ref,
                  y_ref, r_ref):
    r = 1.0 / jnp.maximum(d0_ref[...] + d1_ref[...], 1.0)
    h = (p0_ref[...] + p1_ref[...]) * r + b_ref[...]
    y_ref[...] = jnp.dot(h, w_ref[...], preferred_element_type=jnp.float32)
    r_ref[...] = r


def _tc_norm_mm(p0, p1, d0, d1, b, w):
    return pl.pallas_call(
        _norm_mm_body,
        grid=(N // ROWS_B,),
        in_specs=[pl.BlockSpec((ROWS_B, D), lambda i: (i, 0)),
                  pl.BlockSpec((ROWS_B, D), lambda i: (i, 0)),
                  pl.BlockSpec((ROWS_B, 1), lambda i: (i, 0)),
                  pl.BlockSpec((ROWS_B, 1), lambda i: (i, 0)),
                  pl.BlockSpec((1, D), lambda i: (0, 0)),
                  pl.BlockSpec((D, D), lambda i: (0, 0))],
        out_specs=(pl.BlockSpec((ROWS_B, D), lambda i: (i, 0)),
                   pl.BlockSpec((ROWS_B, 1), lambda i: (i, 0))),
        out_shape=(jax.ShapeDtypeStruct((N, D), jnp.float32),
                   jax.ShapeDtypeStruct((N, 1), jnp.float32)),
    )(p0, p1, d0, d1, b, w)


def _final_body(q0_ref, q1_ref, r_ref, b_ref, o_ref):
    o_ref[...] = (q0_ref[...] + q1_ref[...]) * r_ref[...] + b_ref[...]


def _tc_final(q0, q1, r, b):
    return pl.pallas_call(
        _final_body,
        grid=(N // ROWS_B,),
        in_specs=[pl.BlockSpec((ROWS_B, D), lambda i: (i, 0)),
                  pl.BlockSpec((ROWS_B, D), lambda i: (i, 0)),
                  pl.BlockSpec((ROWS_B, 1), lambda i: (i, 0)),
                  pl.BlockSpec((1, D), lambda i: (0, 0))],
        out_specs=pl.BlockSpec((ROWS_B, D), lambda i: (i, 0)),
        out_shape=jax.ShapeDtypeStruct((N, D), jnp.float32),
    )(q0, q1, r, b)


def kernel(x, edge_index, W1, b1, W2, b2):
    src = edge_index[0].astype(jnp.int32)
    dst = edge_index[1].astype(jnp.int32)
    packed = src | (dst << ISH)
    pidx = jnp.concatenate(
        [packed, jnp.full((EPAD - E,), PAD_DST << ISH, jnp.int32)])

    sc_agg = _make_sc_agg()

    y1 = _tc_mm(x, W1)
    p, dcnt = sc_agg(y1, pidx)
    p0 = p[:N]
    p1 = p[RPAD:RPAD + N]
    d0 = dcnt[:N].reshape(N, 1)
    d1 = dcnt[DPAD:DPAD + N].reshape(N, 1)

    y2, rdeg = _tc_norm_mm(p0, p1, d0, d1, b1.reshape(1, D), W2)

    q, _ = sc_agg(y2, pidx)
    q0 = q[:N]
    q1 = q[RPAD:RPAD + N]
    return _tc_final(q0, q1, rdeg, b2.reshape(1, D))
